# Initial kernel scaffold; baseline (speedup 1.0000x reference)
import jax
import jax.numpy as jnp
from jax.experimental import pallas as pl


def kernel(process_feats, core_states, sys_load, routing_matrix, bias):
    raise NotImplementedError("write your pallas kernel here")


# speedup vs baseline: 3.1650x; 3.1650x over previous
import functools

import jax
import jax.numpy as jnp
from jax.experimental import pallas as pl
from jax.experimental.pallas import tpu as pltpu

BN = 512


def _stats(sx, sxx, n):
    gm = sx * (1.0 / n)
    gv = (sxx - sx * gm) * (1.0 / (n - 1))
    return gv * 0.05 - gm * 0.1


def _p1_kernel(x_ref, sx_ref, sxx_ref):
    nb = pl.program_id(1)
    x = x_ref[0]
    ps = jnp.sum(x, axis=0, keepdims=True)
    pss = jnp.sum(x * x, axis=0, keepdims=True)

    @pl.when(nb == 0)
    def _():
        sx_ref[0] = ps
        sxx_ref[0] = pss

    @pl.when(nb != 0)
    def _():
        sx_ref[0] += ps
        sxx_ref[0] += pss


def _rownorm(y):
    nrm2 = jnp.mean(y * y, axis=1, keepdims=True) + 1e-6
    s = jnp.maximum(jnp.sqrt(nrm2), 1.0)
    return y * (1.0 / s)


def _p2_kernel(n, x_ref, sx_ref, sxx_ref, sx2_ref, sxx2_ref):
    nb = pl.program_id(1)
    w1 = _stats(sx_ref[0], sxx_ref[0], n)
    x1 = _rownorm(x_ref[0] * 1.1 + w1)
    ps = jnp.sum(x1, axis=0, keepdims=True)
    pss = jnp.sum(x1 * x1, axis=0, keepdims=True)

    @pl.when(nb == 0)
    def _():
        sx2_ref[0] = ps
        sxx2_ref[0] = pss

    @pl.when(nb != 0)
    def _():
        sx2_ref[0] += ps
        sxx2_ref[0] += pss


def _p3_kernel(n, x_ref, sx_ref, sxx_ref, sx2_ref, sxx2_ref, rmt_ref,
               bias_ref, proc_ref, d0_ref, d1_ref, v_scr):
    d = x_ref.shape[-1]
    p = bias_ref.shape[-1]
    w1 = _stats(sx_ref[0], sxx_ref[0], n)
    w2 = _stats(sx2_ref[0], sxx2_ref[0], n)
    x1 = _rownorm(x_ref[0] * 1.1 + w1)
    v0 = _rownorm(x1 * 1.1 + w2)
    v_scr[...] = v0

    mu_f0 = jnp.mean(v0[:, : d // 2], axis=1, keepdims=True)
    mu_s0 = jnp.mean(v0[:, d // 2:], axis=1, keepdims=True)
    mu = (mu_f0 + mu_s0) * 0.5
    q = jnp.mean(v0 * v0, axis=1, keepdims=True)
    var = q - mu * mu

    Bc = jnp.zeros_like(mu)
    S = jnp.ones_like(mu)
    for i in range(3):
        c = (mu * 0.15 - var * 0.05) * (0.3 + 0.1 * i)
        q = q + (2.0 * c) * mu + c * c
        mu = mu + c
        Bc = Bc + c * S
        s = jnp.maximum(jnp.sqrt(q + 1e-6), 1.0)
        inv = 1.0 / s
        mu = mu * inv
        q = q * (inv * inv)
        var = var * (inv * inv)
        S = S * s

    invS = 1.0 / S
    mean_all = mu
    var_all = var
    mean_first = (mu_f0 + Bc) * invS
    mean_second = (mu_s0 + Bc) * invS

    x5 = (v_scr[...] + Bc) * invS
    v_scr[...] = x5
    absmean = jnp.mean(jnp.abs(x5), axis=1, keepdims=True)

    dm = jax.lax.dot_general(x5.astype(jnp.bfloat16),
                             rmt_ref[...].astype(jnp.bfloat16),
                             (((1,), (0,)), ((), ())),
                             preferred_element_type=jnp.float32)
    logits = dm + bias_ref[...]

    best = logits[:, 0:1]
    sel = jnp.zeros_like(best, dtype=jnp.int32)
    for j in range(1, p):
        lj = logits[:, j:j + 1]
        upd = lj > best
        best = jnp.where(upd, lj, best)
        sel = jnp.where(upd, j, sel)

    scores = [mean_all, mean_first, mean_second, var_all, -absmean]
    for j in range(5, p):
        scores.append(mean_all * (1.0 + 0.05 * j) - 0.1 * var_all)
    proc = scores[0]
    for j in range(1, p):
        proc = jnp.where(sel == j, scores[j], proc)

    proc_ref[0] = proc
    d0_ref[0] = v_scr[:, 0:1]
    d1_ref[0] = v_scr[:, 1:2]


def _fin_kernel(k, proc_ref, d0_ref, d1_ref, cs_ref, sl_ref, idx_ref, sc_ref):
    b, n = proc_ref.shape
    nc = cs_ref.shape[1]
    cl = jnp.mean(cs_ref[...], axis=2)
    cm = jnp.mean(cl, axis=1, keepdims=True)
    cv = jnp.sum((cl - cm) ** 2, axis=1, keepdims=True) * (1.0 / (nc - 1))
    out = proc_ref[...] + (cm * -0.05 - cv * 0.02)
    ma = jnp.max(jnp.abs(out), axis=1, keepdims=True) + 1e-6
    out = out / jnp.maximum(ma, 1.0)
    delta = d0_ref[...] * 0.05 + d1_ref[...] * 0.03 + sl_ref[:, 0:1] * 0.01
    state = out
    for _ in range(8):
        state = state + delta
    iota = jax.lax.broadcasted_iota(jnp.int32, (b, n), 1)
    for j in range(k):
        m = jnp.max(state, axis=1, keepdims=True)
        idx = jnp.min(jnp.where(state == m, iota, n), axis=1, keepdims=True)
        sc_ref[:, j:j + 1] = m
        idx_ref[:, j:j + 1] = idx
        state = jnp.where(iota == idx, -jnp.inf, state)


def kernel(process_feats, core_states, sys_load, routing_matrix, bias):
    x = process_feats
    b, n, d = x.shape
    p = routing_matrix.shape[0]
    nb = n // BN
    f32 = jnp.float32

    stat = jax.ShapeDtypeStruct((b, 1, d), f32)
    x_spec = pl.BlockSpec((1, BN, d), lambda i, j: (i, j, 0))
    stat_spec = pl.BlockSpec((1, 1, d), lambda i, j: (i, 0, 0))

    sx1, sxx1 = pl.pallas_call(
        _p1_kernel,
        grid=(b, nb),
        in_specs=[x_spec],
        out_specs=[stat_spec, stat_spec],
        out_shape=[stat, stat],
        compiler_params=pltpu.CompilerParams(
            dimension_semantics=("parallel", "arbitrary")),
    )(x)

    sx2, sxx2 = pl.pallas_call(
        functools.partial(_p2_kernel, n),
        grid=(b, nb),
        in_specs=[x_spec, stat_spec, stat_spec],
        out_specs=[stat_spec, stat_spec],
        out_shape=[stat, stat],
        compiler_params=pltpu.CompilerParams(
            dimension_semantics=("parallel", "arbitrary")),
    )(x, sx1, sxx1)

    col = jax.ShapeDtypeStruct((b, n, 1), f32)
    col_spec = pl.BlockSpec((1, BN, 1), lambda i, j: (i, j, 0))
    proc, d0, d1 = pl.pallas_call(
        functools.partial(_p3_kernel, n),
        grid=(b, nb),
        in_specs=[x_spec, stat_spec, stat_spec, stat_spec, stat_spec,
                  pl.BlockSpec((d, p), lambda i, j: (0, 0)),
                  pl.BlockSpec((1, p), lambda i, j: (0, 0))],
        out_specs=[col_spec, col_spec, col_spec],
        out_shape=[col, col, col],
        scratch_shapes=[pltpu.VMEM((BN, d), f32)],
        compiler_params=pltpu.CompilerParams(
            dimension_semantics=("parallel", "parallel")),
    )(x, sx1, sxx1, sx2, sxx2, routing_matrix.T, bias.reshape(1, p))

    k = min(core_states.shape[1], n)
    idx, sc = pl.pallas_call(
        functools.partial(_fin_kernel, k),
        in_specs=[
            pl.BlockSpec((b, n), lambda: (0, 0)),
            pl.BlockSpec((b, n), lambda: (0, 0)),
            pl.BlockSpec((b, n), lambda: (0, 0)),
            pl.BlockSpec(core_states.shape, lambda: (0, 0, 0)),
            pl.BlockSpec(sys_load.shape, lambda: (0, 0)),
        ],
        out_specs=[pl.BlockSpec((b, k), lambda: (0, 0)),
                   pl.BlockSpec((b, k), lambda: (0, 0))],
        out_shape=[jax.ShapeDtypeStruct((b, k), jnp.int32),
                   jax.ShapeDtypeStruct((b, k), f32)],
    )(proc[..., 0], d0[..., 0], d1[..., 0], core_states, sys_load)
    return idx, sc
